# baseline (speedup 1.0000x reference)
import jax
import jax.numpy as jnp
from jax.experimental import pallas as pl

E = 64
DIN = 1024
DOUT = 1024
A = 8
R = 8
T = 2048
GS = T // E
AR = A * R
KSPLIT = 2
KS = DIN // KSPLIT


def _fused_kernel(x_ref, w0_ref, w1_ref, a_ref, b_ref, idx_ref, sc_ref, o_ref):
    g = pl.program_id(0)
    xs = x_ref[pl.ds(g * GS, GS), :]
    acc = jnp.dot(xs[:, 0:KS], w0_ref[0, 0], preferred_element_type=jnp.float32)
    acc += jnp.dot(xs[:, KS:DIN], w1_ref[0, 0], preferred_element_type=jnp.float32)
    xb = xs.astype(jnp.bfloat16)
    idxs = idx_ref[0]
    for a in range(A):
        ia = jnp.dot(xb, a_ref[a, 0].astype(jnp.bfloat16),
                     preferred_element_type=jnp.float32)
        sel = (idxs == a).astype(jnp.float32) * sc_ref[0, a]
        mia = (ia * sel).astype(jnp.bfloat16)
        acc += jnp.dot(mia, b_ref[a, 0].astype(jnp.bfloat16),
                       preferred_element_type=jnp.float32)
    o_ref[...] = acc


def kernel(x, group_sizes, adapter_indices_sorted, weight, lora_A, lora_B, lora_scaling):
    idx = adapter_indices_sorted.reshape(E, GS, 1)
    sc = lora_scaling.reshape(1, A)
    wr = weight.reshape(E, KSPLIT, KS, DOUT)
    out = pl.pallas_call(
        _fused_kernel,
        grid=(E,),
        in_specs=[
            pl.BlockSpec((T, DIN), lambda e: (0, 0)),
            pl.BlockSpec((1, 1, KS, DOUT), lambda e: (e, 0, 0, 0)),
            pl.BlockSpec((1, 1, KS, DOUT), lambda e: (e, 1, 0, 0)),
            pl.BlockSpec((A, 1, DIN, R), lambda e: (0, e, 0, 0)),
            pl.BlockSpec((A, 1, R, DOUT), lambda e: (0, e, 0, 0)),
            pl.BlockSpec((1, GS, 1), lambda e: (e, 0, 0)),
            pl.BlockSpec((1, A), lambda e: (0, 0)),
        ],
        out_specs=pl.BlockSpec((GS, DOUT), lambda e: (e, 0)),
        out_shape=jax.ShapeDtypeStruct((T, DOUT), jnp.float32),
    )(x, wr, wr, lora_A, lora_B, idx, sc)
    return out


# speedup vs baseline: 2.5110x; 2.5110x over previous
import jax
import jax.numpy as jnp
from jax.experimental import pallas as pl

E = 64
DIN = 1024
DOUT = 1024
A = 8
R = 8
T = 2048
GS = T // E
AR = A * R
EPB = 2
NB = E // EPB
KSPLIT = 4
KS = DIN // KSPLIT


def _fused_kernel(x_ref, w0_ref, w1_ref, w2_ref, w3_ref, a_ref, b_ref,
                  idx_ref, o_ref):
    col_adapter = jax.lax.broadcasted_iota(jnp.int32, (GS, AR), 1) // R
    w_refs = (w0_ref, w1_ref, w2_ref, w3_ref)
    for j in range(EPB):
        xs = x_ref[j * GS:(j + 1) * GS, :]
        acc = jnp.dot(xs[:, 0:KS], w0_ref[j, 0],
                      preferred_element_type=jnp.float32)
        for i in range(1, KSPLIT):
            acc += jnp.dot(xs[:, i * KS:(i + 1) * KS], w_refs[i][j, 0],
                           preferred_element_type=jnp.float32)
        inter = jnp.dot(xs.astype(jnp.bfloat16), a_ref[j],
                        preferred_element_type=jnp.float32)
        idxs = idx_ref[0, j * GS:(j + 1) * GS, :]
        mask = (col_adapter == idxs).astype(jnp.float32)
        bmat = b_ref[:, j].reshape(AR, DOUT)
        acc = acc + jnp.dot(inter * mask, bmat, preferred_element_type=jnp.float32)
        o_ref[j * GS:(j + 1) * GS, :] = acc


def kernel(x, group_sizes, adapter_indices_sorted, weight, lora_A, lora_B, lora_scaling):
    a_scaled = lora_A * lora_scaling[:, None, None, None]
    a_stack = a_scaled.transpose(1, 2, 0, 3).reshape(E, DIN, AR).astype(jnp.bfloat16)
    idx = adapter_indices_sorted.reshape(NB, EPB * GS, 1)
    wr = weight.reshape(E, KSPLIT, KS, DOUT)
    w_specs = [
        pl.BlockSpec((EPB, 1, KS, DOUT), lambda g, i=i: (g, i, 0, 0))
        for i in range(KSPLIT)
    ]
    out = pl.pallas_call(
        _fused_kernel,
        grid=(NB,),
        in_specs=[
            pl.BlockSpec((EPB * GS, DIN), lambda g: (g, 0)),
            *w_specs,
            pl.BlockSpec((EPB, DIN, AR), lambda g: (g, 0, 0)),
            pl.BlockSpec((A, EPB, R, DOUT), lambda g: (0, g, 0, 0)),
            pl.BlockSpec((1, EPB * GS, 1), lambda g: (g, 0, 0)),
        ],
        out_specs=pl.BlockSpec((EPB * GS, DOUT), lambda g: (g, 0)),
        out_shape=jax.ShapeDtypeStruct((T, DOUT), jnp.float32),
    )(x, wr, wr, wr, wr, a_stack, lora_B, idx)
    return out
